# baseline (speedup 1.0000x reference)
import dataclasses
import functools

import jax
import jax.numpy as jnp
from jax import lax
from jax.experimental import pallas as pl
from jax.experimental.pallas import tpu as pltpu
from jax.experimental.pallas import tpu_sc as plsc

N = 10000
F = 128
FH = 64
G = 64
NCLS = 40
E = 320000
E_TOT = E + N
N_PAD = 10240
NT = 16
CK = 128
CHUNKS = 168
WCH = 24
EPT = CHUNKS * CK
E_PAD = EPT * NT
RPT = N_PAD // NT
BLK = 1024
NBLK = N_PAD // BLK
NEG = -1e30


def _embed_body(y_ref, w_ref, ats_ref, atd_ref, h_ref, as_ref, ad_ref, m_ref,
                ms_sc, md_sc):
    i = pl.program_id(0)
    h = jnp.dot(y_ref[...], w_ref[...], preferred_element_type=jnp.float32)
    h_ref[0] = h[:, :FH]
    h_ref[1] = h[:, FH:]
    a_s = jnp.sum(h * ats_ref[...], axis=1)
    a_d = jnp.sum(h * atd_ref[...], axis=1)
    rows = i * BLK + lax.broadcasted_iota(jnp.int32, (BLK,), 0)
    valid = rows < N
    a_s = jnp.where(valid, a_s, NEG)
    a_d = jnp.where(valid, a_d, NEG)
    as_ref[...] = a_s.reshape(BLK // 128, 128)
    ad_ref[...] = a_d.reshape(BLK // 128, 128)
    bs = jnp.max(a_s)
    bd = jnp.max(a_d)

    @pl.when(i == 0)
    def _():
        ms_sc[0] = bs
        md_sc[0] = bd

    @pl.when(i > 0)
    def _():
        ms_sc[0] = jnp.maximum(ms_sc[0], bs)
        md_sc[0] = jnp.maximum(md_sc[0], bd)

    mm = ms_sc[0] + md_sc[0]
    mm = jnp.where(mm > 0, mm, 0.2 * mm)
    m_ref[...] = jnp.full((128,), mm, jnp.float32)


def _tc_embed(y, w, ats, atd):
    return pl.pallas_call(
        _embed_body,
        grid=(NBLK,),
        in_specs=[
            pl.BlockSpec((BLK, F), lambda i: (i, 0)),
            pl.BlockSpec((F, F), lambda i: (0, 0)),
            pl.BlockSpec((1, F), lambda i: (0, 0)),
            pl.BlockSpec((1, F), lambda i: (0, 0)),
        ],
        out_specs=[
            pl.BlockSpec((2, BLK, FH), lambda i: (0, i, 0)),
            pl.BlockSpec((BLK // 128, 128), lambda i: (i, 0)),
            pl.BlockSpec((BLK // 128, 128), lambda i: (i, 0)),
            pl.BlockSpec((128,), lambda i: (0,)),
        ],
        out_shape=[
            jax.ShapeDtypeStruct((2, N_PAD, FH), jnp.float32),
            jax.ShapeDtypeStruct((N_PAD // 128, 128), jnp.float32),
            jax.ShapeDtypeStruct((N_PAD // 128, 128), jnp.float32),
            jax.ShapeDtypeStruct((128,), jnp.float32),
        ],
        scratch_shapes=[pltpu.SMEM((1,), jnp.float32),
                        pltpu.SMEM((1,), jnp.float32)],
    )(y, w, ats, atd)


def _bn_elu(num_ref, b_ref, g_ref, be_ref, valid, s1, s2):
    agg = jnp.concatenate([num_ref[0], num_ref[1]], axis=1) + b_ref[...]
    m = s1[...] / N
    v = s2[...] / N - m * m
    xn = (agg - m) * lax.rsqrt(v + 1e-5) * g_ref[...] + be_ref[...]
    el = jnp.where(xn > 0, xn, jnp.exp(jnp.minimum(xn, 0.0)) - 1.0)
    return jnp.where(valid, el, 0.0)


def _bn_stats(num_ref, b_ref, i, valid, s1, s2):
    agg = jnp.concatenate([num_ref[0], num_ref[1]], axis=1) + b_ref[...]
    aggm = jnp.where(valid, agg, 0.0)

    @pl.when(i == 0)
    def _():
        s1[...] = jnp.zeros_like(s1)
        s2[...] = jnp.zeros_like(s2)

    s1[...] += jnp.sum(aggm, axis=0, keepdims=True)
    s2[...] += jnp.sum(aggm * aggm, axis=0, keepdims=True)


def _bn_embed_body(num_ref, b_ref, g_ref, be_ref, w_ref, ats_ref, atd_ref,
                   h_ref, as_ref, ad_ref, m_ref, s1, s2, ms_sc, md_sc):
    p = pl.program_id(0)
    i = pl.program_id(1)
    rows2 = i * BLK + lax.broadcasted_iota(jnp.int32, (BLK, 1), 0)
    valid = rows2 < N

    @pl.when(p == 0)
    def _():
        _bn_stats(num_ref, b_ref, i, valid, s1, s2)

    @pl.when(p == 1)
    def _():
        y = _bn_elu(num_ref, b_ref, g_ref, be_ref, valid, s1, s2)
        h = jnp.dot(y, w_ref[...], preferred_element_type=jnp.float32)
        h_ref[0] = h[:, :FH]
        h_ref[1] = h[:, FH:]
        a_s = jnp.sum(h * ats_ref[...], axis=1)
        a_d = jnp.sum(h * atd_ref[...], axis=1)
        valid1 = i * BLK + lax.broadcasted_iota(jnp.int32, (BLK,), 0) < N
        a_s = jnp.where(valid1, a_s, NEG)
        a_d = jnp.where(valid1, a_d, NEG)
        as_ref[...] = a_s.reshape(BLK // 128, 128)
        ad_ref[...] = a_d.reshape(BLK // 128, 128)
        bs = jnp.max(a_s)
        bd = jnp.max(a_d)

        @pl.when(i == 0)
        def _():
            ms_sc[0] = bs
            md_sc[0] = bd

        @pl.when(i > 0)
        def _():
            ms_sc[0] = jnp.maximum(ms_sc[0], bs)
            md_sc[0] = jnp.maximum(md_sc[0], bd)

        mm = ms_sc[0] + md_sc[0]
        mm = jnp.where(mm > 0, mm, 0.2 * mm)
        m_ref[...] = jnp.full((128,), mm, jnp.float32)


def _tc_bn_embed(num, b, g_, be, w, ats, atd):
    return pl.pallas_call(
        _bn_embed_body,
        grid=(2, NBLK),
        in_specs=[
            pl.BlockSpec((2, BLK, FH), lambda p, i: (0, i, 0)),
            pl.BlockSpec((1, F), lambda p, i: (0, 0)),
            pl.BlockSpec((1, F), lambda p, i: (0, 0)),
            pl.BlockSpec((1, F), lambda p, i: (0, 0)),
            pl.BlockSpec((F, F), lambda p, i: (0, 0)),
            pl.BlockSpec((1, F), lambda p, i: (0, 0)),
            pl.BlockSpec((1, F), lambda p, i: (0, 0)),
        ],
        out_specs=[
            pl.BlockSpec((2, BLK, FH), lambda p, i: (0, i, 0)),
            pl.BlockSpec((BLK // 128, 128), lambda p, i: (i, 0)),
            pl.BlockSpec((BLK // 128, 128), lambda p, i: (i, 0)),
            pl.BlockSpec((128,), lambda p, i: (0,)),
        ],
        out_shape=[
            jax.ShapeDtypeStruct((2, N_PAD, FH), jnp.float32),
            jax.ShapeDtypeStruct((N_PAD // 128, 128), jnp.float32),
            jax.ShapeDtypeStruct((N_PAD // 128, 128), jnp.float32),
            jax.ShapeDtypeStruct((128,), jnp.float32),
        ],
        scratch_shapes=[pltpu.VMEM((1, F), jnp.float32),
                        pltpu.VMEM((1, F), jnp.float32),
                        pltpu.SMEM((1,), jnp.float32),
                        pltpu.SMEM((1,), jnp.float32)],
    )(num, b, g_, be, w, ats, atd)


def _bn_pool_body(num_ref, b_ref, g_ref, be_ref, bat_ref, fw_ref, fb_ref,
                  o_ref, s1, s2, ps, cs):
    p = pl.program_id(0)
    i = pl.program_id(1)
    rows2 = i * BLK + lax.broadcasted_iota(jnp.int32, (BLK, 1), 0)
    valid = rows2 < N

    @pl.when(p == 0)
    def _():
        _bn_stats(num_ref, b_ref, i, valid, s1, s2)

    @pl.when(p == 1)
    def _():
        y = _bn_elu(num_ref, b_ref, g_ref, be_ref, valid, s1, s2)
        seg = lax.broadcasted_iota(jnp.int32, (G, 1), 0).astype(jnp.float32)

        @pl.when(i == 0)
        def _():
            ps[...] = jnp.zeros_like(ps)
            cs[...] = jnp.zeros_like(cs)

        ones = jnp.ones((128, F), jnp.float32)
        for r in range(BLK // 128):
            oh = (bat_ref[pl.ds(r, 1), :] == seg).astype(jnp.float32)
            ps[...] += jnp.dot(oh, y[r * 128:(r + 1) * 128, :],
                               preferred_element_type=jnp.float32)
            cs[...] += jnp.dot(oh, ones, preferred_element_type=jnp.float32)

        @pl.when(i == NBLK - 1)
        def _():
            pooled = ps[...] / jnp.maximum(cs[...], 1.0)
            o_ref[...] = (jnp.dot(pooled, fw_ref[...],
                                  preferred_element_type=jnp.float32) +
                          fb_ref[...])


def _tc_bn_pool(num, b, g_, be, batf, fw, fb):
    return pl.pallas_call(
        _bn_pool_body,
        grid=(2, NBLK),
        in_specs=[
            pl.BlockSpec((2, BLK, FH), lambda p, i: (0, i, 0)),
            pl.BlockSpec((1, F), lambda p, i: (0, 0)),
            pl.BlockSpec((1, F), lambda p, i: (0, 0)),
            pl.BlockSpec((1, F), lambda p, i: (0, 0)),
            pl.BlockSpec((BLK // 128, 128), lambda p, i: (i, 0)),
            pl.BlockSpec((F, NCLS), lambda p, i: (0, 0)),
            pl.BlockSpec((1, NCLS), lambda p, i: (0, 0)),
        ],
        out_specs=pl.BlockSpec((G, NCLS), lambda p, i: (0, 0)),
        out_shape=jax.ShapeDtypeStruct((G, NCLS), jnp.float32),
        scratch_shapes=[pltpu.VMEM((1, F), jnp.float32),
                        pltpu.VMEM((1, F), jnp.float32),
                        pltpu.VMEM((G, F), jnp.float32),
                        pltpu.VMEM((G, F), jnp.float32)],
    )(num, b, g_, be, batf, fw, fb)


def _sc_edge(src3, dst3, a_s, a_d, m8, h):
    mesh = plsc.VectorSubcoreMesh(core_axis_name="c", subcore_axis_name="s")
    cp = pltpu.CompilerParams()
    for fld, val in (("needs_layout_passes", False),
                     ("use_tc_tiling_on_sc", False)):
        if fld in pltpu.CompilerParams.__dataclass_fields__:
            cp = dataclasses.replace(cp, **{fld: val})

    @functools.partial(
        pl.kernel,
        compiler_params=cp,
        out_type=jax.ShapeDtypeStruct((2, N_PAD, FH), jnp.float32),
        mesh=mesh,
        scratch_types=[
            pltpu.VMEM((WCH, CK), jnp.int32),
            pltpu.VMEM((WCH, CK), jnp.int32),
            pltpu.VMEM((WCH, CK), jnp.float32),
            pltpu.VMEM((N_PAD,), jnp.float32),
            pltpu.VMEM((N_PAD,), jnp.float32),
            pltpu.VMEM((128,), jnp.float32),
            pltpu.VMEM((CK, FH), jnp.float32),
            pltpu.VMEM((CK, FH), jnp.float32),
            pltpu.VMEM((RPT,), jnp.float32),
            pltpu.VMEM_SHARED((N_PAD, FH), jnp.float32),
            pltpu.VMEM_SHARED((N_PAD, FH), jnp.float32),
            pltpu.VMEM_SHARED((N_PAD,), jnp.float32),
            pltpu.SemaphoreType.DMA,
            pltpu.SemaphoreType.DMA,
            pltpu.SemaphoreType.DMA,
            pltpu.SemaphoreType.DMA,
        ],
    )
    def k(src_hbm, dst_hbm, as_hbm, ad_hbm, m_hbm, h_hbm,
          num_hbm,
          src_v, dst_v, ex_v, as_v, ad_v, m_v, rows0, rows1, den_t,
          h_sh, acc_sh, den_sh, gsem0, gsem1, ssem0, ssem1):
        c = lax.axis_index("c")
        s = lax.axis_index("s")

        pltpu.sync_copy(as_hbm, as_v)
        pltpu.sync_copy(ad_hbm, ad_v)
        pltpu.sync_copy(m_hbm, m_v)

        base = s * RPT
        pltpu.sync_copy(h_hbm.at[pl.ds(c * N_PAD + base, RPT)],
                        h_sh.at[pl.ds(base, RPT)])

        @pl.loop(0, CK)
        def _(r):
            for kk in range(FH // 16):
                rows0[r, pl.ds(kk * 16, 16)] = jnp.zeros((16,), jnp.float32)

        for j in range(RPT // CK):
            pltpu.sync_copy(rows0, acc_sh.at[pl.ds(base + j * CK, CK)])
        for j in range(RPT // FH):
            pltpu.sync_copy(rows0.at[0], den_sh.at[pl.ds(base + j * FH, FH)])
        plsc.subcore_barrier()

        mvec = m_v[pl.ds(0, 16)]
        gdn = lax.GatherDimensionNumbers(offset_dims=(),
                                         collapsed_slice_dims=(0,),
                                         start_index_map=(0,))

        def bcast(vec, e2):
            idx = jnp.full((16, 1), e2, jnp.int32)
            return lax.gather(vec, idx, gdn, slice_sizes=(1,),
                              mode=lax.GatherScatterMode.PROMISE_IN_BOUNDS)

        def exadj(cur):
            for g_ in range(CK // 16):
                sl = pl.ds(g_ * 16, 16)
                al = (plsc.load_gather(as_v, [src_v[cur, sl]]) +
                      plsc.load_gather(ad_v, [dst_v[cur, sl]]))
                al = jnp.where(al > 0, al, al * 0.2)
                ex_v[cur, sl] = jnp.exp(al - mvec)

        def do_chunk(cur, rows_b, gsem_b, rows_o, gsem_o, ssem_b, ssem_o):
            pltpu.make_async_copy(h_sh.at[src_v.at[cur]], rows_b,
                                  gsem_b).wait()

            @pl.loop(0, CK // 16)
            def _(gq):
                exg = ex_v[cur, pl.ds(gq * 16, 16)]
                for e2 in range(16):
                    bc = bcast(exg, e2)
                    e = gq * 16 + e2
                    for kk in range(FH // 16):
                        sl = pl.ds(kk * 16, 16)
                        rows_b[e, sl] = rows_b[e, sl] * bc

            @pl.when(cur + 2 < WCH)
            def _():
                exadj(cur + 2)

            pltpu.async_copy(rows_b, acc_sh.at[dst_v.at[cur]], ssem_b,
                             add=True)
            pltpu.async_copy(ex_v.at[cur], den_sh.at[dst_v.at[cur]], ssem_b,
                             add=True)

            @pl.when((cur >= 1) & (cur + 1 < WCH))
            def _():
                pltpu.make_async_copy(rows_o, acc_sh.at[dst_v.at[cur]],
                                      ssem_o).wait()
                pltpu.make_async_copy(ex_v.at[cur], den_sh.at[dst_v.at[cur]],
                                      ssem_o).wait()
                pltpu.async_copy(h_sh.at[src_v.at[cur + 1]], rows_o, gsem_o)

        for w in range(CHUNKS // WCH):
            pltpu.sync_copy(src_hbm.at[s, w], src_v)
            pltpu.sync_copy(dst_hbm.at[s, w], dst_v)

            pltpu.async_copy(h_sh.at[src_v.at[0]], rows0, gsem0)
            pltpu.async_copy(h_sh.at[src_v.at[1]], rows1, gsem1)
            exadj(0)
            exadj(1)

            @pl.loop(0, WCH, step=2)
            def _(ci):
                do_chunk(ci, rows0, gsem0, rows1, gsem1, ssem0, ssem1)
                do_chunk(ci + 1, rows1, gsem1, rows0, gsem0, ssem1, ssem0)

            for rows_b, ssem_b in ((rows0, ssem0), (rows1, ssem1)):
                pltpu.make_async_copy(rows_b, acc_sh.at[dst_v.at[0]],
                                      ssem_b).wait()
                pltpu.make_async_copy(ex_v.at[0], den_sh.at[dst_v.at[0]],
                                      ssem_b).wait()

        plsc.subcore_barrier()

        pltpu.sync_copy(den_sh.at[pl.ds(base, RPT)], den_t)
        for j in range(RPT // CK):
            pltpu.sync_copy(acc_sh.at[pl.ds(base + j * CK, CK)], rows0)

            @pl.loop(0, CK)
            def _(r):
                dv = plsc.load_gather(den_t,
                                      [jnp.full((16,), j * CK + r,
                                                jnp.int32)]) + 1e-16
                for kk in range(FH // 16):
                    sl = pl.ds(kk * 16, 16)
                    rows0[r, sl] = rows0[r, sl] / dv

            pltpu.sync_copy(rows0, num_hbm.at[c, pl.ds(base + j * CK, CK)])

    return k(src3, dst3, a_s, a_d, m8, h)


def kernel(x, edge_index, batch, W1, att_src1, att_dst1, b1, g1, be1,
           W2, att_src2, att_dst2, b2, g2, be2, fcW, fcb):
    f32 = jnp.float32
    loop = jnp.arange(N, dtype=edge_index.dtype)
    src = jnp.concatenate([edge_index[0], loop])
    dst = jnp.concatenate([edge_index[1], loop])
    pad = jnp.full((E_PAD - E_TOT,), N, dtype=src.dtype)
    src3 = jnp.concatenate([src, pad]).reshape(NT, CHUNKS // WCH, WCH, CK)
    dst3 = jnp.concatenate([dst, pad]).reshape(NT, CHUNKS // WCH, WCH, CK)
    xp = jnp.zeros((N_PAD, F), f32).at[:N].set(x)
    batf = jnp.full((N_PAD,), G, f32).at[:N].set(batch.astype(f32))
    batf = batf.reshape(N_PAD // 128, 128)

    def edges(as2d, ad2d, m8, h):
        return _sc_edge(src3, dst3, as2d.reshape(N_PAD), ad2d.reshape(N_PAD),
                        m8, h.reshape(2 * N_PAD, FH))

    h1, as1d, ad1d, m1 = _tc_embed(xp, W1, att_src1.reshape(1, F),
                                   att_dst1.reshape(1, F))
    num1 = edges(as1d, ad1d, m1, h1)
    h2, as2d, ad2d, m2 = _tc_bn_embed(num1, b1.reshape(1, F),
                                      g1.reshape(1, F), be1.reshape(1, F),
                                      W2, att_src2.reshape(1, F),
                                      att_dst2.reshape(1, F))
    num2 = edges(as2d, ad2d, m2, h2)
    return _tc_bn_pool(num2, b2.reshape(1, F), g2.reshape(1, F),
                       be2.reshape(1, F), batf, fcW, fcb.reshape(1, NCLS))


# speedup vs baseline: 1.5052x; 1.5052x over previous
import dataclasses
import functools

import jax
import jax.numpy as jnp
from jax import lax
from jax.experimental import pallas as pl
from jax.experimental.pallas import tpu as pltpu
from jax.experimental.pallas import tpu_sc as plsc

N = 10000
F = 128
FH = 64
G = 64
NCLS = 40
E = 320000
E_TOT = E + N
N_PAD = 10240
NT = 16
CK = 128
CHUNKS = 164
WCH = 82
EPT = CHUNKS * CK
E_PAD = EPT * NT
RPT = N_PAD // NT
BLK = 1024
NBLK = N_PAD // BLK
NEG = -1e30


def _embed_body(y_ref, w_ref, ats_ref, atd_ref, h_ref, as_ref, ad_ref, m_ref,
                ms_sc, md_sc):
    i = pl.program_id(0)
    h = jnp.dot(y_ref[...], w_ref[...], preferred_element_type=jnp.float32)
    h_ref[0] = h[:, :FH]
    h_ref[1] = h[:, FH:]
    a_s = jnp.sum(h * ats_ref[...], axis=1)
    a_d = jnp.sum(h * atd_ref[...], axis=1)
    rows = i * BLK + lax.broadcasted_iota(jnp.int32, (BLK,), 0)
    valid = rows < N
    a_s = jnp.where(valid, a_s, NEG)
    a_d = jnp.where(valid, a_d, NEG)
    as_ref[...] = a_s.reshape(BLK // 128, 128)
    ad_ref[...] = a_d.reshape(BLK // 128, 128)
    bs = jnp.max(a_s)
    bd = jnp.max(a_d)

    @pl.when(i == 0)
    def _():
        ms_sc[0] = bs
        md_sc[0] = bd

    @pl.when(i > 0)
    def _():
        ms_sc[0] = jnp.maximum(ms_sc[0], bs)
        md_sc[0] = jnp.maximum(md_sc[0], bd)

    mm = ms_sc[0] + md_sc[0]
    mm = jnp.where(mm > 0, mm, 0.2 * mm)
    m_ref[...] = jnp.full((128,), mm, jnp.float32)


def _tc_embed(y, w, ats, atd):
    return pl.pallas_call(
        _embed_body,
        grid=(NBLK,),
        in_specs=[
            pl.BlockSpec((BLK, F), lambda i: (i, 0)),
            pl.BlockSpec((F, F), lambda i: (0, 0)),
            pl.BlockSpec((1, F), lambda i: (0, 0)),
            pl.BlockSpec((1, F), lambda i: (0, 0)),
        ],
        out_specs=[
            pl.BlockSpec((2, BLK, FH), lambda i: (0, i, 0)),
            pl.BlockSpec((BLK // 128, 128), lambda i: (i, 0)),
            pl.BlockSpec((BLK // 128, 128), lambda i: (i, 0)),
            pl.BlockSpec((128,), lambda i: (0,)),
        ],
        out_shape=[
            jax.ShapeDtypeStruct((2, N_PAD, FH), jnp.float32),
            jax.ShapeDtypeStruct((N_PAD // 128, 128), jnp.float32),
            jax.ShapeDtypeStruct((N_PAD // 128, 128), jnp.float32),
            jax.ShapeDtypeStruct((128,), jnp.float32),
        ],
        scratch_shapes=[pltpu.SMEM((1,), jnp.float32),
                        pltpu.SMEM((1,), jnp.float32)],
    )(y, w, ats, atd)


def _bn_elu(num_ref, b_ref, g_ref, be_ref, valid, s1, s2):
    agg = jnp.concatenate([num_ref[0], num_ref[1]], axis=1) + b_ref[...]
    m = s1[...] / N
    v = s2[...] / N - m * m
    xn = (agg - m) * lax.rsqrt(v + 1e-5) * g_ref[...] + be_ref[...]
    el = jnp.where(xn > 0, xn, jnp.exp(jnp.minimum(xn, 0.0)) - 1.0)
    return jnp.where(valid, el, 0.0)


def _bn_stats(num_ref, b_ref, i, valid, s1, s2):
    agg = jnp.concatenate([num_ref[0], num_ref[1]], axis=1) + b_ref[...]
    aggm = jnp.where(valid, agg, 0.0)

    @pl.when(i == 0)
    def _():
        s1[...] = jnp.zeros_like(s1)
        s2[...] = jnp.zeros_like(s2)

    s1[...] += jnp.sum(aggm, axis=0, keepdims=True)
    s2[...] += jnp.sum(aggm * aggm, axis=0, keepdims=True)


def _bn_embed_body(num_ref, b_ref, g_ref, be_ref, w_ref, ats_ref, atd_ref,
                   h_ref, as_ref, ad_ref, m_ref, s1, s2, ms_sc, md_sc):
    p = pl.program_id(0)
    i = pl.program_id(1)
    rows2 = i * BLK + lax.broadcasted_iota(jnp.int32, (BLK, 1), 0)
    valid = rows2 < N

    @pl.when(p == 0)
    def _():
        _bn_stats(num_ref, b_ref, i, valid, s1, s2)

    @pl.when(p == 1)
    def _():
        y = _bn_elu(num_ref, b_ref, g_ref, be_ref, valid, s1, s2)
        h = jnp.dot(y, w_ref[...], preferred_element_type=jnp.float32)
        h_ref[0] = h[:, :FH]
        h_ref[1] = h[:, FH:]
        a_s = jnp.sum(h * ats_ref[...], axis=1)
        a_d = jnp.sum(h * atd_ref[...], axis=1)
        valid1 = i * BLK + lax.broadcasted_iota(jnp.int32, (BLK,), 0) < N
        a_s = jnp.where(valid1, a_s, NEG)
        a_d = jnp.where(valid1, a_d, NEG)
        as_ref[...] = a_s.reshape(BLK // 128, 128)
        ad_ref[...] = a_d.reshape(BLK // 128, 128)
        bs = jnp.max(a_s)
        bd = jnp.max(a_d)

        @pl.when(i == 0)
        def _():
            ms_sc[0] = bs
            md_sc[0] = bd

        @pl.when(i > 0)
        def _():
            ms_sc[0] = jnp.maximum(ms_sc[0], bs)
            md_sc[0] = jnp.maximum(md_sc[0], bd)

        mm = ms_sc[0] + md_sc[0]
        mm = jnp.where(mm > 0, mm, 0.2 * mm)
        m_ref[...] = jnp.full((128,), mm, jnp.float32)


def _tc_bn_embed(num, b, g_, be, w, ats, atd):
    return pl.pallas_call(
        _bn_embed_body,
        grid=(2, NBLK),
        in_specs=[
            pl.BlockSpec((2, BLK, FH), lambda p, i: (0, i, 0)),
            pl.BlockSpec((1, F), lambda p, i: (0, 0)),
            pl.BlockSpec((1, F), lambda p, i: (0, 0)),
            pl.BlockSpec((1, F), lambda p, i: (0, 0)),
            pl.BlockSpec((F, F), lambda p, i: (0, 0)),
            pl.BlockSpec((1, F), lambda p, i: (0, 0)),
            pl.BlockSpec((1, F), lambda p, i: (0, 0)),
        ],
        out_specs=[
            pl.BlockSpec((2, BLK, FH), lambda p, i: (0, i, 0)),
            pl.BlockSpec((BLK // 128, 128), lambda p, i: (i, 0)),
            pl.BlockSpec((BLK // 128, 128), lambda p, i: (i, 0)),
            pl.BlockSpec((128,), lambda p, i: (0,)),
        ],
        out_shape=[
            jax.ShapeDtypeStruct((2, N_PAD, FH), jnp.float32),
            jax.ShapeDtypeStruct((N_PAD // 128, 128), jnp.float32),
            jax.ShapeDtypeStruct((N_PAD // 128, 128), jnp.float32),
            jax.ShapeDtypeStruct((128,), jnp.float32),
        ],
        scratch_shapes=[pltpu.VMEM((1, F), jnp.float32),
                        pltpu.VMEM((1, F), jnp.float32),
                        pltpu.SMEM((1,), jnp.float32),
                        pltpu.SMEM((1,), jnp.float32)],
    )(num, b, g_, be, w, ats, atd)


def _bn_pool_body(num_ref, b_ref, g_ref, be_ref, bat_ref, fw_ref, fb_ref,
                  o_ref, s1, s2, ps, cs):
    p = pl.program_id(0)
    i = pl.program_id(1)
    rows2 = i * BLK + lax.broadcasted_iota(jnp.int32, (BLK, 1), 0)
    valid = rows2 < N

    @pl.when(p == 0)
    def _():
        _bn_stats(num_ref, b_ref, i, valid, s1, s2)

    @pl.when(p == 1)
    def _():
        y = _bn_elu(num_ref, b_ref, g_ref, be_ref, valid, s1, s2)
        seg = lax.broadcasted_iota(jnp.int32, (G, 1), 0).astype(jnp.float32)

        @pl.when(i == 0)
        def _():
            ps[...] = jnp.zeros_like(ps)
            cs[...] = jnp.zeros_like(cs)

        ones = jnp.ones((128, F), jnp.float32)
        for r in range(BLK // 128):
            oh = (bat_ref[pl.ds(r, 1), :] == seg).astype(jnp.float32)
            ps[...] += jnp.dot(oh, y[r * 128:(r + 1) * 128, :],
                               preferred_element_type=jnp.float32)
            cs[...] += jnp.dot(oh, ones, preferred_element_type=jnp.float32)

        @pl.when(i == NBLK - 1)
        def _():
            pooled = ps[...] / jnp.maximum(cs[...], 1.0)
            o_ref[...] = (jnp.dot(pooled, fw_ref[...],
                                  preferred_element_type=jnp.float32) +
                          fb_ref[...])


def _tc_bn_pool(num, b, g_, be, batf, fw, fb):
    return pl.pallas_call(
        _bn_pool_body,
        grid=(2, NBLK),
        in_specs=[
            pl.BlockSpec((2, BLK, FH), lambda p, i: (0, i, 0)),
            pl.BlockSpec((1, F), lambda p, i: (0, 0)),
            pl.BlockSpec((1, F), lambda p, i: (0, 0)),
            pl.BlockSpec((1, F), lambda p, i: (0, 0)),
            pl.BlockSpec((BLK // 128, 128), lambda p, i: (i, 0)),
            pl.BlockSpec((F, NCLS), lambda p, i: (0, 0)),
            pl.BlockSpec((1, NCLS), lambda p, i: (0, 0)),
        ],
        out_specs=pl.BlockSpec((G, NCLS), lambda p, i: (0, 0)),
        out_shape=jax.ShapeDtypeStruct((G, NCLS), jnp.float32),
        scratch_shapes=[pltpu.VMEM((1, F), jnp.float32),
                        pltpu.VMEM((1, F), jnp.float32),
                        pltpu.VMEM((G, F), jnp.float32),
                        pltpu.VMEM((G, F), jnp.float32)],
    )(num, b, g_, be, batf, fw, fb)


def _sc_edge(src3, dst3, a_s, a_d, m8, h):
    mesh = plsc.VectorSubcoreMesh(core_axis_name="c", subcore_axis_name="s")
    cp = pltpu.CompilerParams()
    for fld, val in (("needs_layout_passes", False),
                     ("use_tc_tiling_on_sc", False)):
        if fld in pltpu.CompilerParams.__dataclass_fields__:
            cp = dataclasses.replace(cp, **{fld: val})

    @functools.partial(
        pl.kernel,
        compiler_params=cp,
        out_type=jax.ShapeDtypeStruct((2, N_PAD, FH), jnp.float32),
        mesh=mesh,
        scratch_types=[
            pltpu.VMEM((WCH, CK), jnp.int32),
            pltpu.VMEM((WCH, CK), jnp.int32),
            pltpu.VMEM((WCH, CK), jnp.float32),
            pltpu.VMEM((N_PAD,), jnp.float32),
            pltpu.VMEM((N_PAD,), jnp.float32),
            pltpu.VMEM((128,), jnp.float32),
            pltpu.VMEM((CK, FH), jnp.float32),
            pltpu.VMEM((CK, FH), jnp.float32),
            pltpu.VMEM((CK, FH), jnp.float32),
            pltpu.VMEM((CK, FH), jnp.float32),
            pltpu.VMEM((RPT,), jnp.float32),
            pltpu.VMEM_SHARED((N_PAD, FH), jnp.float32),
            pltpu.VMEM_SHARED((N_PAD,), jnp.float32),
            pltpu.SemaphoreType.DMA,
            pltpu.SemaphoreType.DMA,
            pltpu.SemaphoreType.DMA,
            pltpu.SemaphoreType.DMA,
        ],
    )
    def k(src_hbm, dst_hbm, as_hbm, ad_hbm, m_hbm, h_hbm,
          num_hbm,
          src_v, dst_v, ex_v, as_v, ad_v, m_v, rows0, rows1, sc0, sc1, den_t,
          acc_sh, den_sh, gsem0, gsem1, ssem0, ssem1):
        c = lax.axis_index("c")
        s = lax.axis_index("s")

        pltpu.sync_copy(as_hbm, as_v)
        pltpu.sync_copy(ad_hbm, ad_v)
        pltpu.sync_copy(m_hbm, m_v)

        @pl.loop(0, CK)
        def _(r):
            for kk in range(FH // 16):
                rows0[r, pl.ds(kk * 16, 16)] = jnp.zeros((16,), jnp.float32)

        base = s * RPT
        for j in range(RPT // CK):
            pltpu.sync_copy(rows0, acc_sh.at[pl.ds(base + j * CK, CK)])
        for j in range(RPT // FH):
            pltpu.sync_copy(rows0.at[0], den_sh.at[pl.ds(base + j * FH, FH)])
        plsc.subcore_barrier()

        mvec = m_v[pl.ds(0, 16)]
        offv = jnp.full((16,), c * N_PAD, jnp.int32)
        gdn = lax.GatherDimensionNumbers(offset_dims=(),
                                         collapsed_slice_dims=(0,),
                                         start_index_map=(0,))

        def bcast(vec, e2):
            idx = jnp.full((16, 1), e2, jnp.int32)
            return lax.gather(vec, idx, gdn, slice_sizes=(1,),
                              mode=lax.GatherScatterMode.PROMISE_IN_BOUNDS)

        def exadj(cur):
            for g_ in range(CK // 16):
                sl = pl.ds(g_ * 16, 16)
                sv = src_v[cur, sl]
                al = (plsc.load_gather(as_v, [sv]) +
                      plsc.load_gather(ad_v, [dst_v[cur, sl]]))
                al = jnp.where(al > 0, al, al * 0.2)
                ex_v[cur, sl] = jnp.exp(al - mvec)
                src_v[cur, sl] = sv + offv

        def do_chunk(cur, rows_b, sc_b, gsem_b, ssem_b):
            pltpu.make_async_copy(h_hbm.at[src_v.at[cur]], rows_b,
                                  gsem_b).wait()

            @pl.when(cur >= 2)
            def _():
                pltpu.make_async_copy(sc_b, acc_sh.at[dst_v.at[cur]],
                                      ssem_b).wait()
                pltpu.make_async_copy(ex_v.at[cur], den_sh.at[dst_v.at[cur]],
                                      ssem_b).wait()

            @pl.loop(0, CK // 16)
            def _(gq):
                exg = ex_v[cur, pl.ds(gq * 16, 16)]
                for e2 in range(16):
                    bc = bcast(exg, e2)
                    e = gq * 16 + e2
                    for kk in range(FH // 16):
                        sl = pl.ds(kk * 16, 16)
                        sc_b[e, sl] = rows_b[e, sl] * bc

            @pl.when(cur + 2 < WCH)
            def _():
                exadj(cur + 2)
                pltpu.async_copy(h_hbm.at[src_v.at[cur + 2]], rows_b, gsem_b)

            pltpu.async_copy(sc_b, acc_sh.at[dst_v.at[cur]], ssem_b,
                             add=True)
            pltpu.async_copy(ex_v.at[cur], den_sh.at[dst_v.at[cur]], ssem_b,
                             add=True)

        for w in range(CHUNKS // WCH):
            pltpu.sync_copy(src_hbm.at[s, w], src_v)
            pltpu.sync_copy(dst_hbm.at[s, w], dst_v)

            exadj(0)
            exadj(1)
            pltpu.async_copy(h_hbm.at[src_v.at[0]], rows0, gsem0)
            pltpu.async_copy(h_hbm.at[src_v.at[1]], rows1, gsem1)

            @pl.loop(0, WCH, step=2)
            def _(ci):
                do_chunk(ci, rows0, sc0, gsem0, ssem0)
                do_chunk(ci + 1, rows1, sc1, gsem1, ssem1)

            for sc_b, ssem_b in ((sc0, ssem0), (sc1, ssem1)):
                pltpu.make_async_copy(sc_b, acc_sh.at[dst_v.at[0]],
                                      ssem_b).wait()
                pltpu.make_async_copy(ex_v.at[0], den_sh.at[dst_v.at[0]],
                                      ssem_b).wait()

        plsc.subcore_barrier()

        pltpu.sync_copy(den_sh.at[pl.ds(base, RPT)], den_t)
        for j in range(RPT // CK):
            pltpu.sync_copy(acc_sh.at[pl.ds(base + j * CK, CK)], rows0)

            @pl.loop(0, CK)
            def _(r):
                dv = plsc.load_gather(den_t,
                                      [jnp.full((16,), j * CK + r,
                                                jnp.int32)]) + 1e-16
                for kk in range(FH // 16):
                    sl = pl.ds(kk * 16, 16)
                    rows0[r, sl] = rows0[r, sl] / dv

            pltpu.sync_copy(rows0, num_hbm.at[c, pl.ds(base + j * CK, CK)])

    return k(src3, dst3, a_s, a_d, m8, h)


def kernel(x, edge_index, batch, W1, att_src1, att_dst1, b1, g1, be1,
           W2, att_src2, att_dst2, b2, g2, be2, fcW, fcb):
    f32 = jnp.float32
    loop = jnp.arange(N, dtype=edge_index.dtype)
    src = jnp.concatenate([edge_index[0], loop])
    dst = jnp.concatenate([edge_index[1], loop])
    pad = jnp.full((E_PAD - E_TOT,), N, dtype=src.dtype)
    src3 = jnp.concatenate([src, pad]).reshape(NT, CHUNKS // WCH, WCH, CK)
    dst3 = jnp.concatenate([dst, pad]).reshape(NT, CHUNKS // WCH, WCH, CK)
    xp = jnp.zeros((N_PAD, F), f32).at[:N].set(x)
    batf = jnp.full((N_PAD,), G, f32).at[:N].set(batch.astype(f32))
    batf = batf.reshape(N_PAD // 128, 128)

    def edges(as2d, ad2d, m8, h):
        return _sc_edge(src3, dst3, as2d.reshape(N_PAD), ad2d.reshape(N_PAD),
                        m8, h.reshape(2 * N_PAD, FH))

    h1, as1d, ad1d, m1 = _tc_embed(xp, W1, att_src1.reshape(1, F),
                                   att_dst1.reshape(1, F))
    num1 = edges(as1d, ad1d, m1, h1)
    h2, as2d, ad2d, m2 = _tc_bn_embed(num1, b1.reshape(1, F),
                                      g1.reshape(1, F), be1.reshape(1, F),
                                      W2, att_src2.reshape(1, F),
                                      att_dst2.reshape(1, F))
    num2 = edges(as2d, ad2d, m2, h2)
    return _tc_bn_pool(num2, b2.reshape(1, F), g2.reshape(1, F),
                       be2.reshape(1, F), batf, fcW, fcb.reshape(1, NCLS))
